# Initial kernel scaffold; baseline (speedup 1.0000x reference)
import jax
import jax.numpy as jnp
from jax.experimental import pallas as pl


def kernel(x, lut):
    raise NotImplementedError("write your pallas kernel here")


# speedup vs baseline: 2.8205x; 2.8205x over previous
import functools
import math

import jax
import jax.numpy as jnp
from jax import lax
from jax.experimental import pallas as pl
from jax.experimental.pallas import tpu as pltpu
from jax.experimental.pallas import tpu_sc as plsc

D_MODEL = 128
SCALE = math.sqrt(float(D_MODEL))
LANES = 16

NUM_CORES = 2
NUM_SUBCORES = 16
NW = NUM_CORES * NUM_SUBCORES

B_TOTAL = 4096 * 50
B_PER_W = B_TOTAL // NW
CHUNK = 128
N_CHUNKS = B_PER_W // CHUNK

_mesh = plsc.VectorSubcoreMesh(core_axis_name="c", subcore_axis_name="s")


def _scale_buf(buf):
    def row_body(r, carry):
        for k in range(D_MODEL // LANES):
            sl = (r, pl.ds(k * LANES, LANES))
            buf[sl] = buf[sl] * SCALE
        return carry

    lax.fori_loop(0, CHUNK, row_body, 0, unroll=False)


@functools.partial(
    pl.kernel,
    out_type=jax.ShapeDtypeStruct((B_TOTAL, D_MODEL), jnp.float32),
    mesh=_mesh,
    scratch_types=[
        pltpu.VMEM((N_CHUNKS, CHUNK), jnp.int32),
        pltpu.VMEM((CHUNK, D_MODEL), jnp.float32),
        pltpu.VMEM((CHUNK, D_MODEL), jnp.float32),
        pltpu.SemaphoreType.DMA,
        pltpu.SemaphoreType.DMA,
        pltpu.SemaphoreType.DMA,
        pltpu.SemaphoreType.DMA,
    ],
)
def _emb_lookup(x_hbm, lut_hbm, out_hbm, idx_v, buf0, buf1, gs0, gs1, os0, os1):
    wid = lax.axis_index("s") * NUM_CORES + lax.axis_index("c")
    base = wid * B_PER_W

    bufs = (buf0, buf1)
    gsems = (gs0, gs1)
    osems = (os0, os1)

    def gather_start(j, bi):
        pltpu.async_copy(lut_hbm.at[idx_v.at[j]], bufs[bi], gsems[bi])

    def gather_wait(j, bi):
        pltpu.make_async_copy(lut_hbm.at[idx_v.at[j]], bufs[bi], gsems[bi]).wait()

    def out_start(j, bi):
        pltpu.async_copy(
            bufs[bi], out_hbm.at[pl.ds(base + j * CHUNK, CHUNK)], osems[bi]
        )

    def out_wait(j, bi):
        pltpu.make_async_copy(
            bufs[bi], out_hbm.at[pl.ds(base + j * CHUNK, CHUNK)], osems[bi]
        ).wait()

    pltpu.sync_copy(x_hbm.at[wid], idx_v)

    gather_start(0, 0)
    gather_start(1, 1)

    gather_wait(0, 0)
    _scale_buf(buf0)
    out_start(0, 0)

    def pair_body(g, carry):
        for b in range(2):
            j = 1 + 2 * g + b
            bi = (1 + b) % 2
            gather_wait(j, bi)
            out_wait(j - 1, 1 - bi)
            gather_start(j + 1, 1 - bi)
            _scale_buf(bufs[bi])
            out_start(j, bi)
        return carry

    lax.fori_loop(0, (N_CHUNKS - 2) // 2, pair_body, 0, unroll=False)

    last = N_CHUNKS - 1
    gather_wait(last, 1)
    out_wait(last - 1, 0)
    _scale_buf(buf1)
    out_start(last, 1)
    out_wait(last, 1)


def kernel(x, lut):
    idx = x.astype(jnp.int32).reshape(NW, N_CHUNKS, CHUNK)
    out = _emb_lookup(idx, lut)
    return out.reshape(x.shape[0], x.shape[1], D_MODEL)


# speedup vs baseline: 2.9447x; 1.0440x over previous
import functools
import math

import jax
import jax.numpy as jnp
from jax import lax
from jax.experimental import pallas as pl
from jax.experimental.pallas import tpu as pltpu
from jax.experimental.pallas import tpu_sc as plsc

D_MODEL = 128
SCALE = math.sqrt(float(D_MODEL))
LANES = 16

NUM_CORES = 2
NUM_SUBCORES = 16
NW = NUM_CORES * NUM_SUBCORES

B_TOTAL = 4096 * 50
B_PER_W = B_TOTAL // NW
CHUNK = 128
N_CHUNKS = B_PER_W // CHUNK
NBUF = 4

_mesh = plsc.VectorSubcoreMesh(core_axis_name="c", subcore_axis_name="s")


def _scale_buf(buf):
    @plsc.parallel_loop(0, CHUNK, step=1, unroll=2)
    def _row(r):
        for k in range(D_MODEL // LANES):
            sl = (r, pl.ds(k * LANES, LANES))
            buf[sl] = buf[sl] * SCALE


@functools.partial(
    pl.kernel,
    out_type=jax.ShapeDtypeStruct((B_TOTAL, D_MODEL), jnp.float32),
    mesh=_mesh,
    scratch_types=[
        pltpu.VMEM((N_CHUNKS, CHUNK), jnp.int32),
        [pltpu.VMEM((CHUNK, D_MODEL), jnp.float32)] * NBUF,
        [pltpu.SemaphoreType.DMA] * NBUF,
        [pltpu.SemaphoreType.DMA] * NBUF,
    ],
)
def _emb_lookup(x_hbm, lut_hbm, out_hbm, idx_v, bufs, gsems, osems):
    wid = lax.axis_index("s") * NUM_CORES + lax.axis_index("c")
    base = wid * B_PER_W

    def gather_start(j, bi):
        pltpu.async_copy(lut_hbm.at[idx_v.at[j]], bufs[bi], gsems[bi])

    def gather_wait(j, bi):
        pltpu.make_async_copy(lut_hbm.at[idx_v.at[j]], bufs[bi], gsems[bi]).wait()

    def out_start(j, bi):
        pltpu.async_copy(
            bufs[bi], out_hbm.at[pl.ds(base + j * CHUNK, CHUNK)], osems[bi]
        )

    def out_wait(j, bi):
        pltpu.make_async_copy(
            bufs[bi], out_hbm.at[pl.ds(base + j * CHUNK, CHUNK)], osems[bi]
        ).wait()

    pltpu.sync_copy(x_hbm.at[wid], idx_v)

    gather_start(0, 0)
    gather_start(1, 1)

    for j in (0, 1):
        gather_wait(j, j % NBUF)
        gather_start(j + 2, (j + 2) % NBUF)
        _scale_buf(bufs[j % NBUF])
        out_start(j, j % NBUF)
    for j in (2, 3):
        gather_wait(j, j % NBUF)
        out_wait(j - 2, (j - 2) % NBUF)
        gather_start(j + 2, (j + 2) % NBUF)
        _scale_buf(bufs[j % NBUF])
        out_start(j, j % NBUF)

    def quad_body(g, carry):
        for b in range(NBUF):
            j = NBUF + NBUF * g + b
            gather_wait(j, b)
            out_wait(j - 2, (b - 2) % NBUF)
            gather_start(j + 2, (b + 2) % NBUF)
            _scale_buf(bufs[b])
            out_start(j, b)
        return carry

    lax.fori_loop(0, (N_CHUNKS - 2 - NBUF) // NBUF, quad_body, 0, unroll=False)

    for j in (N_CHUNKS - 2, N_CHUNKS - 1):
        gather_wait(j, j % NBUF)
        out_wait(j - 2, (j - 2) % NBUF)
        _scale_buf(bufs[j % NBUF])
        out_start(j, j % NBUF)
    out_wait(N_CHUNKS - 2, (N_CHUNKS - 2) % NBUF)
    out_wait(N_CHUNKS - 1, (N_CHUNKS - 1) % NBUF)


def kernel(x, lut):
    idx = x.astype(jnp.int32).reshape(NW, N_CHUNKS, CHUNK)
    out = _emb_lookup(idx, lut)
    return out.reshape(x.shape[0], x.shape[1], D_MODEL)


# speedup vs baseline: 5.1727x; 1.7566x over previous
import functools
import math

import jax
import jax.numpy as jnp
from jax import lax
from jax.experimental import pallas as pl
from jax.experimental.pallas import tpu as pltpu
from jax.experimental.pallas import tpu_sc as plsc

D_MODEL = 128
SCALE = math.sqrt(float(D_MODEL))
LANES = 16

NUM_CORES = 2
NUM_SUBCORES = 16
NW = NUM_CORES * NUM_SUBCORES

N_SEQ = 4096
SEQ_LEN = 50
SEQ_PER_W = N_SEQ // NW
SEQ_PER_CHUNK = 4
N_CHUNKS = SEQ_PER_W // SEQ_PER_CHUNK
NBUF = 4

_mesh = plsc.VectorSubcoreMesh(core_axis_name="c", subcore_axis_name="s")


def _scale_buf(buf):
    @plsc.parallel_loop(0, SEQ_LEN, step=1, unroll=2)
    def _row(t):
        for q in range(SEQ_PER_CHUNK):
            for k in range(D_MODEL // LANES):
                sl = (q, t, pl.ds(k * LANES, LANES))
                buf[sl] = buf[sl] * SCALE


@functools.partial(
    pl.kernel,
    out_type=jax.ShapeDtypeStruct((N_SEQ, SEQ_LEN, D_MODEL), jnp.float32),
    mesh=_mesh,
    compiler_params=pltpu.CompilerParams(use_tc_tiling_on_sc=True),
    scratch_types=[
        pltpu.VMEM((SEQ_PER_W, SEQ_LEN), jnp.int32),
        [pltpu.VMEM((SEQ_PER_CHUNK, SEQ_LEN, D_MODEL), jnp.float32)] * NBUF,
        [pltpu.SemaphoreType.DMA] * NBUF,
        [pltpu.SemaphoreType.DMA] * NBUF,
    ],
)
def _emb_lookup(x_hbm, lut_hbm, out_hbm, idx_v, bufs, gsems, osems):
    wid = lax.axis_index("s") * NUM_CORES + lax.axis_index("c")
    s0 = wid * SEQ_PER_W

    def gather_start(j, bi):
        for q in range(SEQ_PER_CHUNK):
            pltpu.async_copy(
                lut_hbm.at[idx_v.at[j * SEQ_PER_CHUNK + q]],
                bufs[bi].at[q],
                gsems[bi],
            )

    def gather_wait(j, bi):
        for q in range(SEQ_PER_CHUNK):
            pltpu.make_async_copy(
                lut_hbm.at[idx_v.at[j * SEQ_PER_CHUNK + q]],
                bufs[bi].at[q],
                gsems[bi],
            ).wait()

    def out_start(j, bi):
        pltpu.async_copy(
            bufs[bi],
            out_hbm.at[pl.ds(s0 + j * SEQ_PER_CHUNK, SEQ_PER_CHUNK)],
            osems[bi],
        )

    def out_wait(j, bi):
        pltpu.make_async_copy(
            bufs[bi],
            out_hbm.at[pl.ds(s0 + j * SEQ_PER_CHUNK, SEQ_PER_CHUNK)],
            osems[bi],
        ).wait()

    pltpu.sync_copy(x_hbm.at[wid], idx_v)

    gather_start(0, 0)
    gather_start(1, 1)

    for j in (0, 1):
        gather_wait(j, j % NBUF)
        gather_start(j + 2, (j + 2) % NBUF)
        _scale_buf(bufs[j % NBUF])
        out_start(j, j % NBUF)
    for j in (2, 3):
        gather_wait(j, j % NBUF)
        out_wait(j - 2, (j - 2) % NBUF)
        gather_start(j + 2, (j + 2) % NBUF)
        _scale_buf(bufs[j % NBUF])
        out_start(j, j % NBUF)

    def quad_body(g, carry):
        for b in range(NBUF):
            j = NBUF + NBUF * g + b
            gather_wait(j, b)
            out_wait(j - 2, (b - 2) % NBUF)
            gather_start(j + 2, (b + 2) % NBUF)
            _scale_buf(bufs[b])
            out_start(j, b)
        return carry

    lax.fori_loop(0, (N_CHUNKS - 4 - NBUF) // NBUF, quad_body, 0, unroll=False)

    for j in (N_CHUNKS - 4, N_CHUNKS - 3):
        gather_wait(j, j % NBUF)
        out_wait(j - 2, (j - 2) % NBUF)
        gather_start(j + 2, (j + 2) % NBUF)
        _scale_buf(bufs[j % NBUF])
        out_start(j, j % NBUF)
    for j in (N_CHUNKS - 2, N_CHUNKS - 1):
        gather_wait(j, j % NBUF)
        out_wait(j - 2, (j - 2) % NBUF)
        _scale_buf(bufs[j % NBUF])
        out_start(j, j % NBUF)
    out_wait(N_CHUNKS - 2, (N_CHUNKS - 2) % NBUF)
    out_wait(N_CHUNKS - 1, (N_CHUNKS - 1) % NBUF)


def kernel(x, lut):
    idx = x.astype(jnp.int32).reshape(NW, SEQ_PER_W, SEQ_LEN)
    return _emb_lookup(idx, lut)


# speedup vs baseline: 9.3587x; 1.8092x over previous
import functools
import math

import jax
import jax.numpy as jnp
from jax import lax
from jax.experimental import pallas as pl
from jax.experimental.pallas import tpu as pltpu
from jax.experimental.pallas import tpu_sc as plsc

D_MODEL = 128
SCALE = math.sqrt(float(D_MODEL))
LANES = 16

NUM_CORES = 2
NUM_SUBCORES = 16
NW = NUM_CORES * NUM_SUBCORES

N_SEQ = 4096
SEQ_LEN = 50
SEQ_PER_W = N_SEQ // NW
N_CHUNKS = SEQ_LEN
CHUNK = SEQ_PER_W
NBUF = 4

_mesh = plsc.VectorSubcoreMesh(core_axis_name="c", subcore_axis_name="s")


def _scale_buf(buf):
    @plsc.parallel_loop(0, CHUNK, step=1, unroll=2)
    def _row(r):
        for k in range(D_MODEL // LANES):
            sl = (r, pl.ds(k * LANES, LANES))
            buf[sl] = buf[sl] * SCALE


@functools.partial(
    pl.kernel,
    out_type=jax.ShapeDtypeStruct((SEQ_LEN, N_SEQ, D_MODEL), jnp.float32),
    mesh=_mesh,
    compiler_params=pltpu.CompilerParams(use_tc_tiling_on_sc=True),
    scratch_types=[
        pltpu.VMEM((N_CHUNKS, CHUNK), jnp.int32),
        [pltpu.VMEM((CHUNK, D_MODEL), jnp.float32)] * NBUF,
        [pltpu.SemaphoreType.DMA] * NBUF,
        [pltpu.SemaphoreType.DMA] * NBUF,
    ],
)
def _emb_lookup(xt_hbm, lut_hbm, out_hbm, idx_v, bufs, gsems, osems):
    wid = lax.axis_index("s") * NUM_CORES + lax.axis_index("c")
    s0 = wid * SEQ_PER_W

    def gather_start(j, bi):
        pltpu.async_copy(lut_hbm.at[idx_v.at[j]], bufs[bi], gsems[bi])

    def gather_wait(j, bi):
        pltpu.make_async_copy(lut_hbm.at[idx_v.at[j]], bufs[bi], gsems[bi]).wait()

    def out_start(j, bi):
        pltpu.async_copy(bufs[bi], out_hbm.at[j, pl.ds(s0, CHUNK)], osems[bi])

    def out_wait(j, bi):
        pltpu.make_async_copy(
            bufs[bi], out_hbm.at[j, pl.ds(s0, CHUNK)], osems[bi]
        ).wait()

    pltpu.sync_copy(xt_hbm.at[wid], idx_v)

    gather_start(0, 0)
    gather_start(1, 1)

    for j in (0, 1):
        gather_wait(j, j % NBUF)
        gather_start(j + 2, (j + 2) % NBUF)
        _scale_buf(bufs[j % NBUF])
        out_start(j, j % NBUF)
    for j in (2, 3):
        gather_wait(j, j % NBUF)
        out_wait(j - 2, (j - 2) % NBUF)
        gather_start(j + 2, (j + 2) % NBUF)
        _scale_buf(bufs[j % NBUF])
        out_start(j, j % NBUF)

    def quad_body(g, carry):
        for b in range(NBUF):
            j = NBUF + NBUF * g + b
            gather_wait(j, b)
            out_wait(j - 2, (b - 2) % NBUF)
            gather_start(j + 2, (b + 2) % NBUF)
            _scale_buf(bufs[b])
            out_start(j, b)
        return carry

    lax.fori_loop(0, (N_CHUNKS - 2 - NBUF) // NBUF, quad_body, 0, unroll=False)

    for j in (N_CHUNKS - 2, N_CHUNKS - 1):
        gather_wait(j, j % NBUF)
        out_wait(j - 2, (j - 2) % NBUF)
        _scale_buf(bufs[j % NBUF])
        out_start(j, j % NBUF)
    out_wait(N_CHUNKS - 2, (N_CHUNKS - 2) % NBUF)
    out_wait(N_CHUNKS - 1, (N_CHUNKS - 1) % NBUF)


def kernel(x, lut):
    xt = jnp.transpose(
        x.astype(jnp.int32).reshape(NW, SEQ_PER_W, SEQ_LEN), (0, 2, 1)
    )
    out_tmajor = _emb_lookup(xt, lut)
    return jnp.transpose(out_tmajor, (1, 0, 2))


# speedup vs baseline: 9.3614x; 1.0003x over previous
import functools
import math

import jax
import jax.numpy as jnp
from jax import lax
from jax.experimental import pallas as pl
from jax.experimental.pallas import tpu as pltpu
from jax.experimental.pallas import tpu_sc as plsc

D_MODEL = 128
SCALE = math.sqrt(float(D_MODEL))
LANES = 16

NUM_CORES = 2
NUM_SUBCORES = 16
NW = NUM_CORES * NUM_SUBCORES

N_SEQ = 4096
SEQ_LEN = 50
SEQ_PER_W = N_SEQ // NW
N_CHUNKS = SEQ_LEN
CHUNK = SEQ_PER_W
NBUF = 4

_mesh = plsc.VectorSubcoreMesh(core_axis_name="c", subcore_axis_name="s")


def _scale_buf(buf):
    @plsc.parallel_loop(0, CHUNK, step=1, unroll=2)
    def _row(r):
        for k in range(D_MODEL // LANES):
            sl = (r, pl.ds(k * LANES, LANES))
            buf[sl] = buf[sl] * SCALE


@functools.partial(
    pl.kernel,
    out_type=jax.ShapeDtypeStruct((SEQ_LEN, N_SEQ, D_MODEL), jnp.float32),
    mesh=_mesh,
    compiler_params=pltpu.CompilerParams(use_tc_tiling_on_sc=True),
    scratch_types=[
        pltpu.VMEM((N_CHUNKS, CHUNK), jnp.int32),
        [pltpu.VMEM((CHUNK, D_MODEL), jnp.float32)] * NBUF,
        [pltpu.SemaphoreType.DMA] * NBUF,
        [pltpu.SemaphoreType.DMA] * NBUF,
    ],
)
def _emb_lookup(xt_hbm, lut_hbm, out_hbm, idx_v, bufs, gsems, osems):
    wid = lax.axis_index("s") * NUM_CORES + lax.axis_index("c")
    s0 = wid * SEQ_PER_W

    def gather_start(j, bi):
        pltpu.async_copy(lut_hbm.at[idx_v.at[j]], bufs[bi], gsems[bi])

    def gather_wait(j, bi):
        pltpu.make_async_copy(lut_hbm.at[idx_v.at[j]], bufs[bi], gsems[bi]).wait()

    def out_start(j, bi):
        pltpu.async_copy(bufs[bi], out_hbm.at[j, pl.ds(s0, CHUNK)], osems[bi])

    def out_wait(j, bi):
        pltpu.make_async_copy(
            bufs[bi], out_hbm.at[j, pl.ds(s0, CHUNK)], osems[bi]
        ).wait()

    pltpu.sync_copy(xt_hbm.at[:, wid], idx_v)

    gather_start(0, 0)
    gather_start(1, 1)

    for j in (0, 1):
        gather_wait(j, j % NBUF)
        gather_start(j + 2, (j + 2) % NBUF)
        _scale_buf(bufs[j % NBUF])
        out_start(j, j % NBUF)
    for j in (2, 3):
        gather_wait(j, j % NBUF)
        out_wait(j - 2, (j - 2) % NBUF)
        gather_start(j + 2, (j + 2) % NBUF)
        _scale_buf(bufs[j % NBUF])
        out_start(j, j % NBUF)

    def quad_body(g, carry):
        for b in range(NBUF):
            j = NBUF + NBUF * g + b
            gather_wait(j, b)
            out_wait(j - 2, (b - 2) % NBUF)
            gather_start(j + 2, (b + 2) % NBUF)
            _scale_buf(bufs[b])
            out_start(j, b)
        return carry

    lax.fori_loop(0, (N_CHUNKS - 2 - NBUF) // NBUF, quad_body, 0, unroll=False)

    for j in (N_CHUNKS - 2, N_CHUNKS - 1):
        gather_wait(j, j % NBUF)
        out_wait(j - 2, (j - 2) % NBUF)
        _scale_buf(bufs[j % NBUF])
        out_start(j, j % NBUF)
    out_wait(N_CHUNKS - 2, (N_CHUNKS - 2) % NBUF)
    out_wait(N_CHUNKS - 1, (N_CHUNKS - 1) % NBUF)


def kernel(x, lut):
    xt = jnp.transpose(x.astype(jnp.int32), (1, 0)).reshape(
        SEQ_LEN, NW, SEQ_PER_W
    )
    out_tmajor = _emb_lookup(xt, lut)
    return jnp.transpose(out_tmajor, (1, 0, 2))
